# baseline (speedup 1.0000x reference)
import jax
import jax.numpy as jnp
from jax import lax
from jax.experimental import pallas as pl

N = 10000
B = 100
NPG = 100
K = 8
G = 4
NB = G * NPG
EB = NB * K
L1, L2, L3, L4, L5, L6, L7 = 7, 128, 256, 336, 256, 128, 1

_F32 = jnp.float32
_BF16 = jnp.bfloat16


def _lrelu(x):
    return jnp.where(x >= 0, x, 0.01 * x)


def _dotb(a16, b16):
    return lax.dot_general(a16, b16, (((1,), (0,)), ((), ())),
                           preferred_element_type=_F32)


def _dot16(a, w16):
    return _dotb(a.astype(_BF16), w16)


def _knn_onehot(act):
    d2s = []
    for g in range(G):
        pos = act[g * NPG:(g + 1) * NPG, 0:3]
        posT = jnp.transpose(pos)
        t0 = pos[:, 0:1] - posT[0:1, :]
        t1 = pos[:, 1:2] - posT[1:2, :]
        t2 = pos[:, 2:3] - posT[2:3, :]
        d2s.append((t0 * t0 + t1 * t1 + t2 * t2).reshape(1, NPG, NPG))
    d2 = jnp.concatenate(d2s, axis=0)
    rowi = lax.broadcasted_iota(jnp.int32, (G, NPG, NPG), 1)
    colj = lax.broadcasted_iota(jnp.int32, (G, NPG, NPG), 2)
    d2 = jnp.where(rowi == colj, d2 + 1e10, d2)

    rowk = lax.broadcasted_iota(jnp.int32, (G, NPG, K, 1), 2).reshape(G, NPG * K, 1)
    sel_edges = jnp.zeros((G, NPG * K, 1), jnp.int32)
    for k in range(K):
        rmin = jnp.min(d2, axis=2, keepdims=True)
        selk = jnp.min(jnp.where(d2 == rmin, colj, NPG), axis=2, keepdims=True)
        d2 = jnp.where(colj == selk, 1e30, d2)
        selk_rep = jnp.broadcast_to(selk.reshape(G, NPG, 1, 1),
                                    (G, NPG, K, 1)).reshape(G, NPG * K, 1)
        sel_edges = jnp.where(rowk == k, selk_rep, sel_edges)
    goff = lax.broadcasted_iota(jnp.int32, (G, NPG * K, 1), 0) * NPG
    sel_flat = (sel_edges + goff).reshape(EB, 1)
    colj_e = lax.broadcasted_iota(jnp.int32, (EB, NB), 1)
    return (colj_e == sel_flat).astype(_BF16)


def _gather_exact(oh16, act):
    hi = act.astype(_BF16)
    r1 = act - hi.astype(_F32)
    mid = r1.astype(_BF16)
    lo = (r1 - mid.astype(_F32)).astype(_BF16)
    return (_dotb(oh16, hi) + _dotb(oh16, mid)) + _dotb(oh16, lo)


def _rep_edges(x):
    F = x.shape[1]
    return jnp.broadcast_to(x.reshape(NB, 1, F), (NB, K, F)).reshape(EB, F)


def _seg_sum(e):
    e3 = e.reshape(NB, K, e.shape[1])
    s = e3[:, 0]
    for k in range(1, K):
        s = s + e3[:, k]
    return s


def _edge_conv1(act, W1_16, b1, W2_16, b2):
    oh = _knn_onehot(act)
    xj = _gather_exact(oh, act)
    xi = _rep_edges(act)
    m = jnp.concatenate([xi, xj - xi], axis=1)
    h = _lrelu(_dot16(m, W1_16) + b1)
    e = _lrelu(_dot16(h, W2_16) + b2)
    return _seg_sum(e)


def _edge_conv(act, W1a_16, W1b_16, b1, W2_16, b2):
    oh = _knn_onehot(act)
    u = _dot16(act, W1a_16)
    xj = _gather_exact(oh, act)
    dm = _dot16(xj - _rep_edges(act), W1b_16)
    h = _lrelu(_rep_edges(u) + dm + b1)
    e = _lrelu(_dot16(h, W2_16) + b2)
    return _seg_sum(e)


def _body(x_ref, np_ref, mean_ref, scale_ref,
          c1W1, c1b1, c1W2, c1b2,
          c2W1a, c2W1b, c2b1, c2W2, c2b2,
          c3W1a, c3W1b, c3b1, c3W2, c3b2,
          c4W1a, c4W1b, c4b1, c4W2, c4b2,
          n1Wx, n1Wa, n1Wb, n1Wc, n1Wd, n1b_ref,
          n2W_ref, n2b_ref, n3Wm, n3Wnp, n3b_ref, n4W_ref, n4b_ref,
          out_ref):
    xg = x_ref[...]
    xn = (xg - mean_ref[...]) / scale_ref[...]
    a = _edge_conv1(xn, c1W1[...], c1b1[...], c1W2[...], c1b2[...])
    b = _edge_conv(a, c2W1a[...], c2W1b[...], c2b1[...], c2W2[...], c2b2[...])
    c = _edge_conv(b, c3W1a[...], c3W1b[...], c3b1[...], c3W2[...], c3b2[...])
    d = _edge_conv(c, c4W1a[...], c4W1b[...], c4b1[...], c4W2[...], c4b2[...])
    h1 = _lrelu(_dot16(xn, n1Wx[...]) + _dot16(a, n1Wa[...])
                + _dot16(b, n1Wb[...]) + _dot16(c, n1Wc[...])
                + _dot16(d, n1Wd[...]) + n1b_ref[...])
    h2 = _dot16(h1, n2W_ref[...]) + n2b_ref[...]
    pooled_rows = []
    for g in range(G):
        hg = h2[g * NPG:(g + 1) * NPG]
        amax = jnp.max(hg, axis=0, keepdims=True)
        amin = jnp.min(hg, axis=0, keepdims=True)
        asum = jnp.sum(hg, axis=0, keepdims=True)
        pooled_rows.append(jnp.concatenate(
            [amax, amin, asum, asum / float(NPG)], axis=1))
    pq = _lrelu(jnp.concatenate(pooled_rows, axis=0))
    npv = _lrelu(np_ref[...].reshape(G, 1))
    np16 = npv.astype(_BF16).astype(_F32)
    g2 = _lrelu(_dot16(pq, n3Wm[...]) + np16 * n3Wnp[...].astype(_F32)
                + n3b_ref[...])
    y = _dot16(g2, n4W_ref[...]) + n4b_ref[...]
    out_ref[...] = y.reshape(G, 1, 1)


def kernel(x, batch, n_pulses, in_mean, in_scale,
           c1W1, c1b1, c1W2, c1b2, c2W1, c2b1, c2W2, c2b2,
           c3W1, c3b1, c3W2, c3b2, c4W1, c4b1, c4W2, c4b2,
           n1W, n1b, n2W, n2b, n3W, n3b, n4W, n4b):
    np3 = n_pulses.reshape(B, 1, 1)
    row = lambda t: t.reshape(1, -1)
    b16 = lambda t: t.astype(_BF16)
    wspec = pl.BlockSpec(None, lambda i: (0, 0))

    weight_args = [
        b16(c1W1), row(c1b1), b16(c1W2), row(c1b2),
        b16(c2W1[:L3]), b16(c2W1[L3:]), row(c2b1), b16(c2W2), row(c2b2),
        b16(c3W1[:L3]), b16(c3W1[L3:]), row(c3b1), b16(c3W2), row(c3b2),
        b16(c4W1[:L3]), b16(c4W1[L3:]), row(c4b1), b16(c4W2), row(c4b2),
        b16(n1W[:L1]), b16(n1W[L1:L1 + L3]), b16(n1W[L1 + L3:L1 + 2 * L3]),
        b16(n1W[L1 + 2 * L3:L1 + 3 * L3]), b16(n1W[L1 + 3 * L3:]),
        row(n1b), b16(n2W), row(n2b),
        b16(n3W[:4 * L5]), b16(row(n3W[4 * L5])), row(n3b), b16(n4W), row(n4b),
    ]

    out = pl.pallas_call(
        _body,
        grid=(B // G,),
        in_specs=[
            pl.BlockSpec((NB, L1), lambda i: (i, 0)),
            pl.BlockSpec((G, 1, 1), lambda i: (i, 0, 0)),
            pl.BlockSpec(None, lambda i: (0, 0)),
            pl.BlockSpec(None, lambda i: (0, 0)),
        ] + [wspec] * len(weight_args),
        out_specs=pl.BlockSpec((G, 1, 1), lambda i: (i, 0, 0)),
        out_shape=jax.ShapeDtypeStruct((B, 1, 1), _F32),
    )(x, np3, row(in_mean), row(in_scale), *weight_args)
    return out.reshape(B, 1)


# speedup vs baseline: 1.3026x; 1.3026x over previous
import jax
import jax.numpy as jnp
from jax import lax
from jax.experimental import pallas as pl

N = 10000
B = 100
NPG = 100
K = 8
G = 2
NB = G * NPG
EB = NB * K
L1, L2, L3, L4, L5, L6, L7 = 7, 128, 256, 336, 256, 128, 1

_F32 = jnp.float32
_BF16 = jnp.bfloat16


def _lrelu(x):
    return jnp.where(x >= 0, x, 0.01 * x)


def _dotb(a16, b16):
    return lax.dot_general(a16, b16, (((1,), (0,)), ((), ())),
                           preferred_element_type=_F32)


def _dot16(a, w16):
    return _dotb(a.astype(_BF16), w16)


def _knn_onehot(act):
    d2s = []
    for g in range(G):
        pos = act[g * NPG:(g + 1) * NPG, 0:3]
        posT = jnp.transpose(pos)
        t0 = pos[:, 0:1] - posT[0:1, :]
        t1 = pos[:, 1:2] - posT[1:2, :]
        t2 = pos[:, 2:3] - posT[2:3, :]
        d2s.append((t0 * t0 + t1 * t1 + t2 * t2).reshape(1, NPG, NPG))
    d2 = jnp.concatenate(d2s, axis=0)
    rowi = lax.broadcasted_iota(jnp.int32, (G, NPG, NPG), 1)
    colj = lax.broadcasted_iota(jnp.int32, (G, NPG, NPG), 2)
    d2 = jnp.where(rowi == colj, d2 + 1e10, d2)

    rowk = lax.broadcasted_iota(jnp.int32, (G, NPG, K, 1), 2).reshape(G, NPG * K, 1)
    sel_edges = jnp.zeros((G, NPG * K, 1), jnp.int32)
    for k in range(K):
        rmin = jnp.min(d2, axis=2, keepdims=True)
        selk = jnp.min(jnp.where(d2 == rmin, colj, NPG), axis=2, keepdims=True)
        d2 = jnp.where(colj == selk, 1e30, d2)
        selk_rep = jnp.broadcast_to(selk.reshape(G, NPG, 1, 1),
                                    (G, NPG, K, 1)).reshape(G, NPG * K, 1)
        sel_edges = jnp.where(rowk == k, selk_rep, sel_edges)
    goff = lax.broadcasted_iota(jnp.int32, (G, NPG * K, 1), 0) * NPG
    sel_flat = (sel_edges + goff).reshape(EB, 1)
    colj_e = lax.broadcasted_iota(jnp.int32, (EB, NB), 1)
    return (colj_e == sel_flat).astype(_BF16)


def _gather_exact(oh16, act):
    hi = act.astype(_BF16)
    r1 = act - hi.astype(_F32)
    mid = r1.astype(_BF16)
    lo = (r1 - mid.astype(_F32)).astype(_BF16)
    return (_dotb(oh16, hi) + _dotb(oh16, mid)) + _dotb(oh16, lo)


def _rep_edges(x):
    F = x.shape[1]
    return jnp.broadcast_to(x.reshape(NB, 1, F), (NB, K, F)).reshape(EB, F)


def _seg_sum(e):
    e3 = e.reshape(NB, K, e.shape[1])
    s = e3[:, 0]
    for k in range(1, K):
        s = s + e3[:, k]
    return s


def _edge_conv1(act, W1_16, b1, W2_16, b2):
    oh = _knn_onehot(act)
    xj = _gather_exact(oh, act)
    xi = _rep_edges(act)
    m = jnp.concatenate([xi, xj - xi], axis=1)
    h = _lrelu(_dot16(m, W1_16) + b1)
    e = _lrelu(_dot16(h, W2_16) + b2)
    return _seg_sum(e)


def _edge_conv(act, W1a_16, W1b_16, b1, W2_16, b2):
    oh = _knn_onehot(act)
    u = _dot16(act, W1a_16)
    xj = _gather_exact(oh, act)
    dm = _dot16(xj - _rep_edges(act), W1b_16)
    h = _lrelu(_rep_edges(u) + dm + b1)
    e = _lrelu(_dot16(h, W2_16) + b2)
    return _seg_sum(e)


def _body(x_ref, np_ref, mean_ref, scale_ref,
          c1W1, c1b1, c1W2, c1b2,
          c2W1a, c2W1b, c2b1, c2W2, c2b2,
          c3W1a, c3W1b, c3b1, c3W2, c3b2,
          c4W1a, c4W1b, c4b1, c4W2, c4b2,
          n1Wx, n1Wa, n1Wb, n1Wc, n1Wd, n1b_ref,
          n2W_ref, n2b_ref, n3Wm, n3Wnp, n3b_ref, n4W_ref, n4b_ref,
          out_ref):
    xg = x_ref[...]
    xn = (xg - mean_ref[...]) / scale_ref[...]
    a = _edge_conv1(xn, c1W1[...], c1b1[...], c1W2[...], c1b2[...])
    b = _edge_conv(a, c2W1a[...], c2W1b[...], c2b1[...], c2W2[...], c2b2[...])
    c = _edge_conv(b, c3W1a[...], c3W1b[...], c3b1[...], c3W2[...], c3b2[...])
    d = _edge_conv(c, c4W1a[...], c4W1b[...], c4b1[...], c4W2[...], c4b2[...])
    h1 = _lrelu(_dot16(xn, n1Wx[...]) + _dot16(a, n1Wa[...])
                + _dot16(b, n1Wb[...]) + _dot16(c, n1Wc[...])
                + _dot16(d, n1Wd[...]) + n1b_ref[...])
    h2 = _dot16(h1, n2W_ref[...]) + n2b_ref[...]
    pooled_rows = []
    for g in range(G):
        hg = h2[g * NPG:(g + 1) * NPG]
        amax = jnp.max(hg, axis=0, keepdims=True)
        amin = jnp.min(hg, axis=0, keepdims=True)
        asum = jnp.sum(hg, axis=0, keepdims=True)
        pooled_rows.append(jnp.concatenate(
            [amax, amin, asum, asum / float(NPG)], axis=1))
    pq = _lrelu(jnp.concatenate(pooled_rows, axis=0))
    npv = _lrelu(np_ref[...].reshape(G, 1))
    np16 = npv.astype(_BF16).astype(_F32)
    g2 = _lrelu(_dot16(pq, n3Wm[...]) + np16 * n3Wnp[...].astype(_F32)
                + n3b_ref[...])
    y = _dot16(g2, n4W_ref[...]) + n4b_ref[...]
    out_ref[...] = y.reshape(G, 1, 1)


def kernel(x, batch, n_pulses, in_mean, in_scale,
           c1W1, c1b1, c1W2, c1b2, c2W1, c2b1, c2W2, c2b2,
           c3W1, c3b1, c3W2, c3b2, c4W1, c4b1, c4W2, c4b2,
           n1W, n1b, n2W, n2b, n3W, n3b, n4W, n4b):
    np3 = n_pulses.reshape(B, 1, 1)
    row = lambda t: t.reshape(1, -1)
    b16 = lambda t: t.astype(_BF16)
    wspec = pl.BlockSpec(None, lambda i: (0, 0))

    weight_args = [
        b16(c1W1), row(c1b1), b16(c1W2), row(c1b2),
        b16(c2W1[:L3]), b16(c2W1[L3:]), row(c2b1), b16(c2W2), row(c2b2),
        b16(c3W1[:L3]), b16(c3W1[L3:]), row(c3b1), b16(c3W2), row(c3b2),
        b16(c4W1[:L3]), b16(c4W1[L3:]), row(c4b1), b16(c4W2), row(c4b2),
        b16(n1W[:L1]), b16(n1W[L1:L1 + L3]), b16(n1W[L1 + L3:L1 + 2 * L3]),
        b16(n1W[L1 + 2 * L3:L1 + 3 * L3]), b16(n1W[L1 + 3 * L3:]),
        row(n1b), b16(n2W), row(n2b),
        b16(n3W[:4 * L5]), b16(row(n3W[4 * L5])), row(n3b), b16(n4W), row(n4b),
    ]

    out = pl.pallas_call(
        _body,
        grid=(B // G,),
        in_specs=[
            pl.BlockSpec((NB, L1), lambda i: (i, 0)),
            pl.BlockSpec((G, 1, 1), lambda i: (i, 0, 0)),
            pl.BlockSpec(None, lambda i: (0, 0)),
            pl.BlockSpec(None, lambda i: (0, 0)),
        ] + [wspec] * len(weight_args),
        out_specs=pl.BlockSpec((G, 1, 1), lambda i: (i, 0, 0)),
        out_shape=jax.ShapeDtypeStruct((B, 1, 1), _F32),
    )(x, np3, row(in_mean), row(in_scale), *weight_args)
    return out.reshape(B, 1)
